# Initial kernel scaffold; baseline (speedup 1.0000x reference)
import jax
import jax.numpy as jnp
from jax.experimental import pallas as pl


def kernel(atomic_numbers_in, species_table):
    raise NotImplementedError("write your pallas kernel here")


# speedup vs baseline: 117.0564x; 117.0564x over previous
import functools

import jax
import jax.numpy as jnp
from jax import lax
from jax.experimental import pallas as pl
from jax.experimental.pallas import tpu as pltpu
from jax.experimental.pallas import tpu_sc as plsc

N_NODES = 100000
N_SPECIES = 119
TAB_PAD = 128

NUM_CORES = 2
NUM_SUBCORES = 16
NW = NUM_CORES * NUM_SUBCORES

CHUNK = 3136
LAST = N_NODES - (NW - 1) * CHUNK


@functools.partial(
    pl.kernel,
    out_type=jax.ShapeDtypeStruct((N_NODES,), jnp.int32),
    mesh=plsc.VectorSubcoreMesh(core_axis_name="c", subcore_axis_name="s"),
    scratch_types=[
        pltpu.VMEM((TAB_PAD,), jnp.int32),
        pltpu.VMEM((TAB_PAD,), jnp.int32),
        pltpu.VMEM_SHARED((TAB_PAD,), jnp.int32),
        pltpu.VMEM((CHUNK,), jnp.int32),
        pltpu.VMEM((CHUNK,), jnp.int32),
    ],
)
def _species_lookup(a_hbm, tab_hbm, out_hbm, tab_v, jidx_v, inv_sh, in_v, res_v):
    c = lax.axis_index("c")
    s = lax.axis_index("s")
    wid = s * NUM_CORES + c

    @pl.when(s == 0)
    def _():
        pltpu.sync_copy(tab_hbm, tab_v)
        lane = lax.iota(jnp.int32, 16)
        for j in range(TAB_PAD // 16):
            jidx_v[pl.ds(j * 16, 16)] = lane + (j * 16)
        pltpu.sync_copy(jidx_v, inv_sh.at[tab_v])

    plsc.subcore_barrier()

    def run(base, nelem):
        pltpu.sync_copy(a_hbm.at[pl.ds(base, nelem)], in_v.at[pl.ds(0, nelem)])
        pltpu.sync_copy(
            inv_sh.at[in_v.at[pl.ds(0, nelem)]], res_v.at[pl.ds(0, nelem)]
        )
        pltpu.sync_copy(res_v.at[pl.ds(0, nelem)], out_hbm.at[pl.ds(base, nelem)])

    @pl.when(wid < NW - 1)
    def _():
        run(wid * CHUNK, CHUNK)

    @pl.when(wid == NW - 1)
    def _():
        run((NW - 1) * CHUNK, LAST)


def kernel(atomic_numbers_in, species_table):
    a = atomic_numbers_in.astype(jnp.int32)
    tab = jnp.concatenate(
        [species_table.astype(jnp.int32),
         jnp.arange(N_SPECIES, TAB_PAD, dtype=jnp.int32)]
    )
    return _species_lookup(a, tab)


# speedup vs baseline: 118.9209x; 1.0159x over previous
import functools

import jax
import jax.numpy as jnp
from jax import lax
from jax.experimental import pallas as pl
from jax.experimental.pallas import tpu as pltpu
from jax.experimental.pallas import tpu_sc as plsc

N_NODES = 100000
N_SPECIES = 119
TAB_PAD = 128

NUM_CORES = 2
NUM_SUBCORES = 16
NW = NUM_CORES * NUM_SUBCORES

CHUNK = 3136
LAST = N_NODES - (NW - 1) * CHUNK


@functools.partial(
    pl.kernel,
    out_type=jax.ShapeDtypeStruct((N_NODES,), jnp.int32),
    mesh=plsc.VectorSubcoreMesh(core_axis_name="c", subcore_axis_name="s"),
    scratch_types=[
        pltpu.VMEM((TAB_PAD,), jnp.int32),
        pltpu.VMEM((TAB_PAD,), jnp.int32),
        pltpu.VMEM_SHARED((TAB_PAD,), jnp.int32),
        pltpu.VMEM((CHUNK,), jnp.int32),
        pltpu.VMEM((CHUNK,), jnp.int32),
        pltpu.SemaphoreType.DMA,
    ],
)
def _species_lookup(
    a_hbm, tab_hbm, out_hbm, tab_v, jidx_v, inv_sh, in_v, res_v, sem
):
    c = lax.axis_index("c")
    s = lax.axis_index("s")
    wid = s * NUM_CORES + c

    @pl.when(s == 0)
    def _():
        cp = pltpu.async_copy(
            a_hbm.at[pl.ds(wid * CHUNK, CHUNK)], in_v.at[pl.ds(0, CHUNK)], sem
        )
        pltpu.sync_copy(tab_hbm, tab_v)
        lane = lax.iota(jnp.int32, 16)
        for j in range(TAB_PAD // 16):
            jidx_v[pl.ds(j * 16, 16)] = lane + (j * 16)
        pltpu.sync_copy(jidx_v, inv_sh.at[tab_v])
        cp.wait()

    @pl.when((s != 0) & (wid < NW - 1))
    def _():
        pltpu.sync_copy(
            a_hbm.at[pl.ds(wid * CHUNK, CHUNK)], in_v.at[pl.ds(0, CHUNK)]
        )

    @pl.when(wid == NW - 1)
    def _():
        pltpu.sync_copy(
            a_hbm.at[pl.ds((NW - 1) * CHUNK, LAST)], in_v.at[pl.ds(0, LAST)]
        )

    plsc.subcore_barrier()

    def translate(base, nelem):
        pltpu.sync_copy(
            inv_sh.at[in_v.at[pl.ds(0, nelem)]], res_v.at[pl.ds(0, nelem)]
        )
        pltpu.sync_copy(res_v.at[pl.ds(0, nelem)], out_hbm.at[pl.ds(base, nelem)])

    @pl.when(wid < NW - 1)
    def _():
        translate(wid * CHUNK, CHUNK)

    @pl.when(wid == NW - 1)
    def _():
        translate((NW - 1) * CHUNK, LAST)


def kernel(atomic_numbers_in, species_table):
    a = atomic_numbers_in.astype(jnp.int32)
    tab = jnp.concatenate(
        [species_table.astype(jnp.int32),
         jnp.arange(N_SPECIES, TAB_PAD, dtype=jnp.int32)]
    )
    return _species_lookup(a, tab)


# speedup vs baseline: 119.1529x; 1.0020x over previous
import functools

import jax
import jax.numpy as jnp
from jax import lax
from jax.experimental import pallas as pl
from jax.experimental.pallas import tpu as pltpu
from jax.experimental.pallas import tpu_sc as plsc

N_NODES = 100000
N_SPECIES = 119
TAB_PAD = 128

NUM_CORES = 2
NUM_SUBCORES = 16
NW = NUM_CORES * NUM_SUBCORES

CHUNK = 3136
LAST = N_NODES - (NW - 1) * CHUNK


@functools.partial(
    pl.kernel,
    out_type=jax.ShapeDtypeStruct((N_NODES,), jnp.int32),
    mesh=plsc.VectorSubcoreMesh(core_axis_name="c", subcore_axis_name="s"),
    scratch_types=[
        pltpu.VMEM((TAB_PAD,), jnp.int32),
        pltpu.VMEM((TAB_PAD,), jnp.int32),
        pltpu.VMEM_SHARED((TAB_PAD,), jnp.int32),
        pltpu.VMEM((CHUNK,), jnp.int32),
        pltpu.VMEM((CHUNK,), jnp.int32),
        pltpu.SemaphoreType.DMA,
    ],
)
def _species_lookup(
    a_hbm, tab_hbm, out_hbm, tab_v, jidx_v, inv_sh, in_v, res_v, sem
):
    c = lax.axis_index("c")
    s = lax.axis_index("s")
    wid = s * NUM_CORES + c

    @pl.when(s == 0)
    def _():
        cp = pltpu.async_copy(
            a_hbm.at[pl.ds(wid * CHUNK, CHUNK)], in_v.at[pl.ds(0, CHUNK)], sem
        )
        pltpu.sync_copy(tab_hbm, tab_v.at[pl.ds(0, N_SPECIES)])
        lane = lax.iota(jnp.int32, 16)
        tail_base = (TAB_PAD // 16 - 1) * 16
        tail = tab_v[pl.ds(tail_base, 16)]
        tail_lane = lane + tail_base
        tab_v[pl.ds(tail_base, 16)] = jnp.where(
            tail_lane < N_SPECIES, tail, tail_lane
        )
        for j in range(TAB_PAD // 16):
            jidx_v[pl.ds(j * 16, 16)] = lane + (j * 16)
        pltpu.sync_copy(jidx_v, inv_sh.at[tab_v])
        cp.wait()

    @pl.when((s != 0) & (wid < NW - 1))
    def _():
        pltpu.sync_copy(
            a_hbm.at[pl.ds(wid * CHUNK, CHUNK)], in_v.at[pl.ds(0, CHUNK)]
        )

    @pl.when(wid == NW - 1)
    def _():
        pltpu.sync_copy(
            a_hbm.at[pl.ds((NW - 1) * CHUNK, LAST)], in_v.at[pl.ds(0, LAST)]
        )

    plsc.subcore_barrier()

    def translate(base, nelem):
        pltpu.sync_copy(
            inv_sh.at[in_v.at[pl.ds(0, nelem)]], res_v.at[pl.ds(0, nelem)]
        )
        pltpu.sync_copy(res_v.at[pl.ds(0, nelem)], out_hbm.at[pl.ds(base, nelem)])

    @pl.when(wid < NW - 1)
    def _():
        translate(wid * CHUNK, CHUNK)

    @pl.when(wid == NW - 1)
    def _():
        translate((NW - 1) * CHUNK, LAST)


def kernel(atomic_numbers_in, species_table):
    a = atomic_numbers_in.astype(jnp.int32)
    tab = species_table.astype(jnp.int32)
    return _species_lookup(a, tab)
